# baseline (speedup 1.0000x reference)
import dataclasses
import functools

import jax
import jax.numpy as jnp
from jax import lax
from jax.experimental import pallas as pl
from jax.experimental.pallas import tpu as pltpu
from jax.experimental.pallas import tpu_sc as plsc

N = 10000
NP = 10240
E = 320000
E_PAD = 327680
LBL = 4096
D1 = 128
D2 = 64
NC, NS, L = 2, 16, 16
NW = NC * NS
WE = 128
RPT = NP // NS
GW = 80


def _vmesh():
    return plsc.VectorSubcoreMesh(core_axis_name="c", subcore_axis_name="s")


def _no_layout_cp():
    cp = pltpu.CompilerParams()
    if "needs_layout_passes" in pltpu.CompilerParams.__dataclass_fields__:
        cp = dataclasses.replace(cp, needs_layout_passes=False)
    return cp


def _sc_pre(table, nid_pad, dst2w, ew2w, zrow):
    nwin = (E_PAD // NW) // WE
    ng_w = NP // NW
    ngw = ng_w // GW

    @functools.partial(
        pl.kernel,
        out_type=(jax.ShapeDtypeStruct((NP, D1), jnp.float32),
                  jax.ShapeDtypeStruct((NC, NP), jnp.float32)),
        mesh=_vmesh(),
        scratch_types=[
            pltpu.VMEM_SHARED((NP,), jnp.float32),
            pltpu.VMEM((GW, D1), jnp.float32),
            pltpu.VMEM((1, GW), jnp.int32),
            pltpu.VMEM((nwin, WE), jnp.int32),
            pltpu.VMEM((nwin, WE), jnp.float32),
            pltpu.SemaphoreType.DMA,
        ],
    )
    def k(table_hbm, nid_hbm, dst_hbm, ew_hbm, zrow_hbm, x_hbm, deg_hbm,
          deg_acc, rows, gbuf, dbuf, ebuf, dsem):
        c = lax.axis_index("c")
        s = lax.axis_index("s")
        w = c * NS + s

        pltpu.sync_copy(zrow_hbm, deg_acc.at[pl.ds(s * RPT, RPT)])
        pltpu.sync_copy(dst_hbm.at[pl.ds(w * nwin, nwin)], dbuf)
        pltpu.sync_copy(ew_hbm.at[pl.ds(w * nwin, nwin)], ebuf)
        plsc.subcore_barrier()

        @pl.loop(0, nwin)
        def _(j):
            pltpu.make_async_copy(ebuf.at[j], deg_acc.at[dbuf.at[j]],
                                  dsem).start(add=True)

        @pl.loop(0, ngw)
        def _(j):
            base = w * ng_w + j * GW
            pltpu.sync_copy(nid_hbm.at[pl.ds(base, GW)], gbuf.at[0])
            pltpu.sync_copy(table_hbm.at[gbuf.at[0]], rows)
            pltpu.sync_copy(rows, x_hbm.at[pl.ds(base, GW)])

        @pl.loop(0, nwin)
        def _(j):
            pltpu.make_async_copy(ebuf.at[j], deg_acc.at[dbuf.at[j]],
                                  dsem).wait()

        plsc.subcore_barrier()
        pltpu.sync_copy(deg_acc.at[pl.ds(s * RPT, RPT)],
                        deg_hbm.at[c, pl.ds(s * RPT, RPT)])

    return k(table, nid_pad, dst2w, ew2w, zrow)


WEA = 128
NWIN_A = (E_PAD // NW) // WEA
CWIN = 40
NW0, NW1 = 160, 0


def _sc_agg(hs, src2d, dst2d, ew2d, zblk, d_eff):
    @functools.partial(
        pl.kernel,
        out_type=jax.ShapeDtypeStruct((NP, D1), jnp.float32),
        mesh=_vmesh(),
        compiler_params=_no_layout_cp(),
        scratch_types=[
            pltpu.VMEM_SHARED((NP, D1), jnp.float32),
            pltpu.VMEM((CWIN, WEA), jnp.int32),
            pltpu.VMEM((CWIN, WEA), jnp.int32),
            pltpu.VMEM((CWIN, WEA), jnp.float32),
            pltpu.VMEM((WEA, D1), jnp.float32),
            pltpu.VMEM((WEA, D1), jnp.float32),
            pltpu.SemaphoreType.DMA,
            pltpu.SemaphoreType.DMA,
            pltpu.SemaphoreType.DMA,
            pltpu.SemaphoreType.DMA,
        ],
    )
    def k(hs_hbm, src_hbm, dst_hbm, ew_hbm, zblk_hbm, out_hbm,
          acc, sbuf, dbuf, ebuf, in0, in1, g0, g1, s0, s1):
        c = lax.axis_index("c")
        s = lax.axis_index("s")
        w = c * NS + s

        def scale(buf, erow):
            @pl.loop(0, WEA, step=4)
            def _(i):
                for u in range(4):
                    iu = i + u
                    es = plsc.load_gather(
                        erow, [jnp.full((L,), iu, jnp.int32)])
                    for kk in range(d_eff // L):
                        sl = pl.ds(kk * L, L)
                        buf[iu, sl] = buf[iu, sl] * es

        @pl.when(c == 0)
        def _():
            pltpu.sync_copy(zblk_hbm, acc.at[pl.ds(s * RPT, RPT)])
        plsc.subcore_barrier()

        nch = jnp.where(c == 0, NW0 // CWIN, NW1 // CWIN)
        woff = s * NW0

        @pl.loop(0, nch)
        def _(ch):
            cbase = woff + ch * CWIN
            pltpu.sync_copy(src_hbm.at[pl.ds(cbase, CWIN)], sbuf)
            pltpu.sync_copy(dst_hbm.at[pl.ds(cbase, CWIN)], dbuf)
            pltpu.sync_copy(ew_hbm.at[pl.ds(cbase, CWIN)], ebuf)

            pltpu.make_async_copy(hs_hbm.at[sbuf.at[0]], in0, g0).start()

            @pl.loop(0, CWIN, step=2)
            def _(j):
                @pl.when(j > 0)
                def _():
                    pltpu.make_async_copy(in1, acc.at[dbuf.at[j - 1]],
                                          s1).wait()
                pltpu.make_async_copy(hs_hbm.at[sbuf.at[j + 1]], in1,
                                      g1).start()

                pltpu.make_async_copy(hs_hbm.at[sbuf.at[j]], in0, g0).wait()
                scale(in0, ebuf.at[j])
                pltpu.make_async_copy(in0, acc.at[dbuf.at[j]],
                                      s0).start(add=True)

                pltpu.make_async_copy(hs_hbm.at[sbuf.at[j + 1]], in1,
                                      g1).wait()
                scale(in1, ebuf.at[j + 1])

                pltpu.make_async_copy(in0, acc.at[dbuf.at[j]], s0).wait()

                @pl.when(j + 2 < CWIN)
                def _():
                    pltpu.make_async_copy(hs_hbm.at[sbuf.at[j + 2]], in0,
                                          g0).start()

                pltpu.make_async_copy(in1, acc.at[dbuf.at[j + 1]],
                                      s1).start(add=True)

            pltpu.make_async_copy(in1, acc.at[dbuf.at[CWIN - 1]], s1).wait()

        plsc.subcore_barrier()

        @pl.when(c == 0)
        def _():
            pltpu.sync_copy(acc.at[pl.ds(s * RPT, RPT)],
                            out_hbm.at[pl.ds(s * RPT, RPT)])

    return k(hs, src2d, dst2d, ew2d, zblk)


def _sc_decode(z, si, di):
    pw = LBL // NW

    @functools.partial(
        pl.kernel,
        out_type=(jax.ShapeDtypeStruct((LBL, D1), jnp.float32),
                  jax.ShapeDtypeStruct((LBL, D1), jnp.float32)),
        mesh=_vmesh(),
        scratch_types=[
            pltpu.VMEM((LBL // NW, D1), jnp.float32),
            pltpu.VMEM((1, LBL // NW), jnp.int32),
        ],
    )
    def k(z_hbm, si_hbm, di_hbm, zs_hbm, zd_hbm, rows, ibuf):
        c = lax.axis_index("c")
        s = lax.axis_index("s")
        base = (c * NS + s) * pw

        pltpu.sync_copy(si_hbm.at[pl.ds(base, pw)], ibuf.at[0])
        pltpu.sync_copy(z_hbm.at[ibuf.at[0]], rows)
        pltpu.sync_copy(rows, zs_hbm.at[pl.ds(base, pw)])

        pltpu.sync_copy(di_hbm.at[pl.ds(base, pw)], ibuf.at[0])
        pltpu.sync_copy(z_hbm.at[ibuf.at[0]], rows)
        pltpu.sync_copy(rows, zd_hbm.at[pl.ds(base, pw)])

    return k(z, si, di)


_BR = 512


def _dis_from(deg_ref):
    return lax.rsqrt(deg_ref[:, 0:1] + deg_ref[:, 1:2] + 1.0)


def _tc_scale_matmul(x, deg_t, w1):
    def body(x_ref, deg_ref, w_ref, o_ref):
        dis = _dis_from(deg_ref)
        h = jnp.dot(x_ref[...], w_ref[...], preferred_element_type=jnp.float32)
        o_ref[...] = h * dis

    return pl.pallas_call(
        body,
        grid=(NP // _BR,),
        in_specs=[pl.BlockSpec((_BR, D1), lambda i: (i, 0)),
                  pl.BlockSpec((_BR, NC), lambda i: (i, 0)),
                  pl.BlockSpec((D1, D1), lambda i: (0, 0))],
        out_specs=pl.BlockSpec((_BR, D1), lambda i: (i, 0)),
        out_shape=jax.ShapeDtypeStruct((NP, D1), jnp.float32),
    )(x, deg_t, w1)


def _tc_mid(agg1, hs1, deg_t, b1, w2):
    def body(agg_ref, hs_ref, deg_ref, b_ref, w_ref, o_ref):
        dis = _dis_from(deg_ref)
        tot = agg_ref[...] + hs_ref[...]
        h = jnp.maximum(tot * dis + b_ref[...], 0.0)
        hs2 = jnp.dot(h, w_ref[...], preferred_element_type=jnp.float32) * dis
        o_ref[...] = jnp.concatenate(
            [hs2, jnp.zeros((_BR, D1 - D2), jnp.float32)], axis=1)

    return pl.pallas_call(
        body,
        grid=(NP // _BR,),
        in_specs=[pl.BlockSpec((_BR, D1), lambda i: (i, 0)),
                  pl.BlockSpec((_BR, D1), lambda i: (i, 0)),
                  pl.BlockSpec((_BR, NC), lambda i: (i, 0)),
                  pl.BlockSpec((1, D1), lambda i: (0, 0)),
                  pl.BlockSpec((D1, D2), lambda i: (0, 0))],
        out_specs=pl.BlockSpec((_BR, D1), lambda i: (i, 0)),
        out_shape=jax.ShapeDtypeStruct((NP, D1), jnp.float32),
    )(agg1, hs1, deg_t, b1, w2)


def _tc_final(agg2, hs2, deg_t, b2):
    def body(agg_ref, hs_ref, deg_ref, b_ref, o_ref):
        dis = _dis_from(deg_ref)
        tot = agg_ref[...] + hs_ref[...]
        o_ref[...] = tot * dis + b_ref[...]

    return pl.pallas_call(
        body,
        grid=(NP // _BR,),
        in_specs=[pl.BlockSpec((_BR, D1), lambda i: (i, 0)),
                  pl.BlockSpec((_BR, D1), lambda i: (i, 0)),
                  pl.BlockSpec((_BR, NC), lambda i: (i, 0)),
                  pl.BlockSpec((1, D1), lambda i: (0, 0))],
        out_specs=pl.BlockSpec((_BR, D1), lambda i: (i, 0)),
        out_shape=jax.ShapeDtypeStruct((NP, D1), jnp.float32),
    )(agg2, hs2, deg_t, b2)


def _tc_dot(zs, zd):
    def body(a_ref, b_ref, o_ref):
        o_ref[...] = jnp.sum(a_ref[:, :D2] * b_ref[:, :D2], axis=1,
                             keepdims=True)

    return pl.pallas_call(
        body,
        grid=(LBL // 1024,),
        in_specs=[pl.BlockSpec((1024, D1), lambda i: (i, 0)),
                  pl.BlockSpec((1024, D1), lambda i: (i, 0))],
        out_specs=pl.BlockSpec((1024, 1), lambda i: (i, 0)),
        out_shape=jax.ShapeDtypeStruct((LBL, 1), jnp.float32),
    )(zs, zd)


def kernel(n_id, edge_index, edge_attr, edge_label_index, table, W1, b1, W2, b2):
    i32 = jnp.int32
    nid_pad = jnp.concatenate(
        [n_id.astype(i32), jnp.zeros((NP - N,), i32)])
    src_pad = jnp.concatenate(
        [edge_index[0].astype(i32), jnp.zeros((E_PAD - E,), i32)])
    dst_pad = jnp.concatenate(
        [edge_index[1].astype(i32), jnp.zeros((E_PAD - E,), i32)])
    ew_pad = jnp.concatenate(
        [edge_attr.astype(jnp.float32), jnp.zeros((E_PAD - E,), jnp.float32)])
    si = edge_label_index[0].astype(i32)
    di = edge_label_index[1].astype(i32)

    zrow = jnp.zeros((RPT,), jnp.float32)
    zblk1 = jnp.zeros((RPT, D1), jnp.float32)
    b2p = jnp.concatenate([b2.astype(jnp.float32),
                           jnp.zeros((D1 - D2,), jnp.float32)]).reshape(1, D1)

    src2d = src_pad.reshape(E_PAD // WEA, WEA)
    dst2d = dst_pad.reshape(E_PAD // WEA, WEA)
    ew2d = ew_pad.reshape(E_PAD // WEA, WEA)

    x, deg2 = _sc_pre(table, nid_pad, dst_pad.reshape(E_PAD // WE, WE),
                      ew_pad.reshape(E_PAD // WE, WE), zrow)
    deg_t = deg2.T

    hs1 = _tc_scale_matmul(x, deg_t, W1)
    agg1 = _sc_agg(hs1, src2d, dst2d, ew2d, zblk1, D1)
    hs2 = _tc_mid(agg1, hs1, deg_t, b1.reshape(1, D1), W2)
    agg2 = _sc_agg(hs2, src2d, dst2d, ew2d, zblk1, D2)
    z = _tc_final(agg2, hs2, deg_t, b2p)

    zs, zd = _sc_decode(z, si, di)
    return _tc_dot(zs, zd).reshape(LBL)


# speedup vs baseline: 1.3227x; 1.3227x over previous
import dataclasses
import functools

import jax
import jax.numpy as jnp
from jax import lax
from jax.experimental import pallas as pl
from jax.experimental.pallas import tpu as pltpu
from jax.experimental.pallas import tpu_sc as plsc

N = 10000
NP = 10240
E = 320000
E_PAD = 327680
LBL = 4096
D1 = 128
D2 = 64
NC, NS, L = 2, 16, 16
NW = NC * NS
WE = 128
RPT = NP // NS
GW = 80


def _vmesh():
    return plsc.VectorSubcoreMesh(core_axis_name="c", subcore_axis_name="s")


def _no_layout_cp():
    cp = pltpu.CompilerParams()
    if "needs_layout_passes" in pltpu.CompilerParams.__dataclass_fields__:
        cp = dataclasses.replace(cp, needs_layout_passes=False)
    return cp


def _sc_pre(table, nid_pad, dst2w, ew2w, zrow):
    nwin = (E_PAD // NW) // WE
    ng_w = NP // NW
    ngw = ng_w // GW

    @functools.partial(
        pl.kernel,
        out_type=(jax.ShapeDtypeStruct((NP, D1), jnp.float32),
                  jax.ShapeDtypeStruct((NC, NP), jnp.float32)),
        mesh=_vmesh(),
        scratch_types=[
            pltpu.VMEM_SHARED((NP,), jnp.float32),
            pltpu.VMEM((GW, D1), jnp.float32),
            pltpu.VMEM((1, GW), jnp.int32),
            pltpu.VMEM((nwin, WE), jnp.int32),
            pltpu.VMEM((nwin, WE), jnp.float32),
            pltpu.SemaphoreType.DMA,
        ],
    )
    def k(table_hbm, nid_hbm, dst_hbm, ew_hbm, zrow_hbm, x_hbm, deg_hbm,
          deg_acc, rows, gbuf, dbuf, ebuf, dsem):
        c = lax.axis_index("c")
        s = lax.axis_index("s")
        w = c * NS + s

        pltpu.sync_copy(zrow_hbm, deg_acc.at[pl.ds(s * RPT, RPT)])
        pltpu.sync_copy(dst_hbm.at[pl.ds(w * nwin, nwin)], dbuf)
        pltpu.sync_copy(ew_hbm.at[pl.ds(w * nwin, nwin)], ebuf)
        plsc.subcore_barrier()

        @pl.loop(0, nwin)
        def _(j):
            pltpu.make_async_copy(ebuf.at[j], deg_acc.at[dbuf.at[j]],
                                  dsem).start(add=True)

        @pl.loop(0, ngw)
        def _(j):
            base = w * ng_w + j * GW
            pltpu.sync_copy(nid_hbm.at[pl.ds(base, GW)], gbuf.at[0])
            pltpu.sync_copy(table_hbm.at[gbuf.at[0]], rows)
            pltpu.sync_copy(rows, x_hbm.at[pl.ds(base, GW)])

        @pl.loop(0, nwin)
        def _(j):
            pltpu.make_async_copy(ebuf.at[j], deg_acc.at[dbuf.at[j]],
                                  dsem).wait()

        plsc.subcore_barrier()
        pltpu.sync_copy(deg_acc.at[pl.ds(s * RPT, RPT)],
                        deg_hbm.at[c, pl.ds(s * RPT, RPT)])

    return k(table, nid_pad, dst2w, ew2w, zrow)


WEA = 128
NWIN_A = (E_PAD // NW) // WEA
CWIN = 40
NW0, NW1 = 120, 40


def _sc_agg(hs, src2d, dst2d, ew2d, zblk, d_eff):
    @functools.partial(
        pl.kernel,
        out_type=jax.ShapeDtypeStruct((NC, NP, D1), jnp.float32),
        mesh=_vmesh(),
        compiler_params=_no_layout_cp(),
        scratch_types=[
            pltpu.VMEM_SHARED((NP, D1), jnp.float32),
            pltpu.VMEM((CWIN, WEA), jnp.int32),
            pltpu.VMEM((CWIN, WEA), jnp.int32),
            pltpu.VMEM((CWIN, WEA), jnp.float32),
            pltpu.VMEM((WEA, D1), jnp.float32),
            pltpu.VMEM((WEA, D1), jnp.float32),
            pltpu.SemaphoreType.DMA,
            pltpu.SemaphoreType.DMA,
            pltpu.SemaphoreType.DMA,
            pltpu.SemaphoreType.DMA,
        ],
    )
    def k(hs_hbm, src_hbm, dst_hbm, ew_hbm, zblk_hbm, out_hbm,
          acc, sbuf, dbuf, ebuf, in0, in1, g0, g1, s0, s1):
        c = lax.axis_index("c")
        s = lax.axis_index("s")
        w = c * NS + s

        def scale(buf, erow):
            @pl.loop(0, WEA, step=4)
            def _(i):
                for u in range(4):
                    iu = i + u
                    es = plsc.load_gather(
                        erow, [jnp.full((L,), iu, jnp.int32)])
                    for kk in range(d_eff // L):
                        sl = pl.ds(kk * L, L)
                        buf[iu, sl] = buf[iu, sl] * es

        pltpu.sync_copy(zblk_hbm.at[c], acc.at[pl.ds(s * RPT, RPT)])
        plsc.subcore_barrier()

        nch = jnp.where(c == 0, NW0 // CWIN, NW1 // CWIN)
        woff = jnp.where(c == 0, s * NW0, NS * NW0 + s * NW1)

        @pl.loop(0, nch)
        def _(ch):
            cbase = woff + ch * CWIN
            pltpu.sync_copy(src_hbm.at[pl.ds(cbase, CWIN)], sbuf)
            pltpu.sync_copy(dst_hbm.at[pl.ds(cbase, CWIN)], dbuf)
            pltpu.sync_copy(ew_hbm.at[pl.ds(cbase, CWIN)], ebuf)

            pltpu.make_async_copy(hs_hbm.at[sbuf.at[0]], in0, g0).start()

            @pl.loop(0, CWIN, step=2)
            def _(j):
                @pl.when(j > 0)
                def _():
                    pltpu.make_async_copy(in1, acc.at[dbuf.at[j - 1]],
                                          s1).wait()
                pltpu.make_async_copy(hs_hbm.at[sbuf.at[j + 1]], in1,
                                      g1).start()

                pltpu.make_async_copy(hs_hbm.at[sbuf.at[j]], in0, g0).wait()
                scale(in0, ebuf.at[j])
                pltpu.make_async_copy(in0, acc.at[dbuf.at[j]],
                                      s0).start(add=True)

                pltpu.make_async_copy(hs_hbm.at[sbuf.at[j + 1]], in1,
                                      g1).wait()
                scale(in1, ebuf.at[j + 1])

                pltpu.make_async_copy(in0, acc.at[dbuf.at[j]], s0).wait()

                @pl.when(j + 2 < CWIN)
                def _():
                    pltpu.make_async_copy(hs_hbm.at[sbuf.at[j + 2]], in0,
                                          g0).start()

                pltpu.make_async_copy(in1, acc.at[dbuf.at[j + 1]],
                                      s1).start(add=True)

            pltpu.make_async_copy(in1, acc.at[dbuf.at[CWIN - 1]], s1).wait()

        plsc.subcore_barrier()
        pltpu.sync_copy(acc.at[pl.ds(s * RPT, RPT)],
                        out_hbm.at[c, pl.ds(s * RPT, RPT)])

    return k(hs, src2d, dst2d, ew2d, zblk)


def _sc_decode(z, si, di):
    pw = LBL // NW

    @functools.partial(
        pl.kernel,
        out_type=(jax.ShapeDtypeStruct((LBL, D1), jnp.float32),
                  jax.ShapeDtypeStruct((LBL, D1), jnp.float32)),
        mesh=_vmesh(),
        scratch_types=[
            pltpu.VMEM((LBL // NW, D1), jnp.float32),
            pltpu.VMEM((1, LBL // NW), jnp.int32),
        ],
    )
    def k(z_hbm, si_hbm, di_hbm, zs_hbm, zd_hbm, rows, ibuf):
        c = lax.axis_index("c")
        s = lax.axis_index("s")
        base = (c * NS + s) * pw

        pltpu.sync_copy(si_hbm.at[pl.ds(base, pw)], ibuf.at[0])
        pltpu.sync_copy(z_hbm.at[ibuf.at[0]], rows)
        pltpu.sync_copy(rows, zs_hbm.at[pl.ds(base, pw)])

        pltpu.sync_copy(di_hbm.at[pl.ds(base, pw)], ibuf.at[0])
        pltpu.sync_copy(z_hbm.at[ibuf.at[0]], rows)
        pltpu.sync_copy(rows, zd_hbm.at[pl.ds(base, pw)])

    return k(z, si, di)


_BR = 512


def _dis_from(deg_ref):
    return lax.rsqrt(deg_ref[:, 0:1] + deg_ref[:, 1:2] + 1.0)


def _tc_scale_matmul(x, deg_t, w1):
    def body(x_ref, deg_ref, w_ref, o_ref):
        dis = _dis_from(deg_ref)
        h = jnp.dot(x_ref[...], w_ref[...], preferred_element_type=jnp.float32)
        o_ref[...] = h * dis

    return pl.pallas_call(
        body,
        grid=(NP // _BR,),
        in_specs=[pl.BlockSpec((_BR, D1), lambda i: (i, 0)),
                  pl.BlockSpec((_BR, NC), lambda i: (i, 0)),
                  pl.BlockSpec((D1, D1), lambda i: (0, 0))],
        out_specs=pl.BlockSpec((_BR, D1), lambda i: (i, 0)),
        out_shape=jax.ShapeDtypeStruct((NP, D1), jnp.float32),
    )(x, deg_t, w1)


def _tc_mid(agg1, hs1, deg_t, b1, w2):
    def body(agg_ref, hs_ref, deg_ref, b_ref, w_ref, o_ref):
        dis = _dis_from(deg_ref)
        tot = agg_ref[0] + agg_ref[1] + hs_ref[...]
        h = jnp.maximum(tot * dis + b_ref[...], 0.0)
        hs2 = jnp.dot(h, w_ref[...], preferred_element_type=jnp.float32) * dis
        o_ref[...] = jnp.concatenate(
            [hs2, jnp.zeros((_BR, D1 - D2), jnp.float32)], axis=1)

    return pl.pallas_call(
        body,
        grid=(NP // _BR,),
        in_specs=[pl.BlockSpec((NC, _BR, D1), lambda i: (0, i, 0)),
                  pl.BlockSpec((_BR, D1), lambda i: (i, 0)),
                  pl.BlockSpec((_BR, NC), lambda i: (i, 0)),
                  pl.BlockSpec((1, D1), lambda i: (0, 0)),
                  pl.BlockSpec((D1, D2), lambda i: (0, 0))],
        out_specs=pl.BlockSpec((_BR, D1), lambda i: (i, 0)),
        out_shape=jax.ShapeDtypeStruct((NP, D1), jnp.float32),
    )(agg1, hs1, deg_t, b1, w2)


def _tc_final(agg2, hs2, deg_t, b2):
    def body(agg_ref, hs_ref, deg_ref, b_ref, o_ref):
        dis = _dis_from(deg_ref)
        tot = agg_ref[0] + agg_ref[1] + hs_ref[...]
        o_ref[...] = tot * dis + b_ref[...]

    return pl.pallas_call(
        body,
        grid=(NP // _BR,),
        in_specs=[pl.BlockSpec((NC, _BR, D1), lambda i: (0, i, 0)),
                  pl.BlockSpec((_BR, D1), lambda i: (i, 0)),
                  pl.BlockSpec((_BR, NC), lambda i: (i, 0)),
                  pl.BlockSpec((1, D1), lambda i: (0, 0))],
        out_specs=pl.BlockSpec((_BR, D1), lambda i: (i, 0)),
        out_shape=jax.ShapeDtypeStruct((NP, D1), jnp.float32),
    )(agg2, hs2, deg_t, b2)


def _tc_dot(zs, zd):
    def body(a_ref, b_ref, o_ref):
        o_ref[...] = jnp.sum(a_ref[:, :D2] * b_ref[:, :D2], axis=1,
                             keepdims=True)

    return pl.pallas_call(
        body,
        grid=(LBL // 1024,),
        in_specs=[pl.BlockSpec((1024, D1), lambda i: (i, 0)),
                  pl.BlockSpec((1024, D1), lambda i: (i, 0))],
        out_specs=pl.BlockSpec((1024, 1), lambda i: (i, 0)),
        out_shape=jax.ShapeDtypeStruct((LBL, 1), jnp.float32),
    )(zs, zd)


def kernel(n_id, edge_index, edge_attr, edge_label_index, table, W1, b1, W2, b2):
    i32 = jnp.int32
    nid_pad = jnp.concatenate(
        [n_id.astype(i32), jnp.zeros((NP - N,), i32)])
    src_pad = jnp.concatenate(
        [edge_index[0].astype(i32), jnp.zeros((E_PAD - E,), i32)])
    dst_pad = jnp.concatenate(
        [edge_index[1].astype(i32), jnp.zeros((E_PAD - E,), i32)])
    ew_pad = jnp.concatenate(
        [edge_attr.astype(jnp.float32), jnp.zeros((E_PAD - E,), jnp.float32)])
    si = edge_label_index[0].astype(i32)
    di = edge_label_index[1].astype(i32)

    zrow = jnp.zeros((RPT,), jnp.float32)
    zblk1 = jnp.zeros((NC, RPT, D1), jnp.float32)
    b2p = jnp.concatenate([b2.astype(jnp.float32),
                           jnp.zeros((D1 - D2,), jnp.float32)]).reshape(1, D1)

    src2d = src_pad.reshape(E_PAD // WEA, WEA)
    dst2d = dst_pad.reshape(E_PAD // WEA, WEA)
    ew2d = ew_pad.reshape(E_PAD // WEA, WEA)

    x, deg2 = _sc_pre(table, nid_pad, dst_pad.reshape(E_PAD // WE, WE),
                      ew_pad.reshape(E_PAD // WE, WE), zrow)
    deg_t = deg2.T

    hs1 = _tc_scale_matmul(x, deg_t, W1)
    agg1 = _sc_agg(hs1, src2d, dst2d, ew2d, zblk1, D1)
    hs2 = _tc_mid(agg1, hs1, deg_t, b1.reshape(1, D1), W2)
    agg2 = _sc_agg(hs2, src2d, dst2d, ew2d, zblk1, D2)
    z = _tc_final(agg2, hs2, deg_t, b2p)

    zs, zd = _sc_decode(z, si, di)
    return _tc_dot(zs, zd).reshape(LBL)


# speedup vs baseline: 1.3236x; 1.0007x over previous
import dataclasses
import functools

import jax
import jax.numpy as jnp
from jax import lax
from jax.experimental import pallas as pl
from jax.experimental.pallas import tpu as pltpu
from jax.experimental.pallas import tpu_sc as plsc

N = 10000
NP = 10240
E = 320000
E_PAD = 327680
LBL = 4096
D1 = 128
D2 = 64
NC, NS, L = 2, 16, 16
NW = NC * NS
WE = 128
RPT = NP // NS
GW = 80


def _vmesh():
    return plsc.VectorSubcoreMesh(core_axis_name="c", subcore_axis_name="s")


def _no_layout_cp():
    cp = pltpu.CompilerParams()
    if "needs_layout_passes" in pltpu.CompilerParams.__dataclass_fields__:
        cp = dataclasses.replace(cp, needs_layout_passes=False)
    return cp


def _sc_pre(table, nid_pad, dst2w, ew2w, zrow):
    nwin = (E_PAD // NW) // WE
    ng_w = NP // NW
    ngw = ng_w // GW

    @functools.partial(
        pl.kernel,
        out_type=(jax.ShapeDtypeStruct((NP, D1), jnp.float32),
                  jax.ShapeDtypeStruct((NC, NP), jnp.float32)),
        mesh=_vmesh(),
        scratch_types=[
            pltpu.VMEM_SHARED((NP,), jnp.float32),
            pltpu.VMEM((GW, D1), jnp.float32),
            pltpu.VMEM((1, GW), jnp.int32),
            pltpu.VMEM((nwin, WE), jnp.int32),
            pltpu.VMEM((nwin, WE), jnp.float32),
            pltpu.SemaphoreType.DMA,
        ],
    )
    def k(table_hbm, nid_hbm, dst_hbm, ew_hbm, zrow_hbm, x_hbm, deg_hbm,
          deg_acc, rows, gbuf, dbuf, ebuf, dsem):
        c = lax.axis_index("c")
        s = lax.axis_index("s")
        w = c * NS + s

        pltpu.sync_copy(zrow_hbm, deg_acc.at[pl.ds(s * RPT, RPT)])
        pltpu.sync_copy(dst_hbm.at[pl.ds(w * nwin, nwin)], dbuf)
        pltpu.sync_copy(ew_hbm.at[pl.ds(w * nwin, nwin)], ebuf)
        plsc.subcore_barrier()

        @pl.loop(0, nwin)
        def _(j):
            pltpu.make_async_copy(ebuf.at[j], deg_acc.at[dbuf.at[j]],
                                  dsem).start(add=True)

        @pl.loop(0, ngw)
        def _(j):
            base = w * ng_w + j * GW
            pltpu.sync_copy(nid_hbm.at[pl.ds(base, GW)], gbuf.at[0])
            pltpu.sync_copy(table_hbm.at[gbuf.at[0]], rows)
            pltpu.sync_copy(rows, x_hbm.at[pl.ds(base, GW)])

        @pl.loop(0, nwin)
        def _(j):
            pltpu.make_async_copy(ebuf.at[j], deg_acc.at[dbuf.at[j]],
                                  dsem).wait()

        plsc.subcore_barrier()
        pltpu.sync_copy(deg_acc.at[pl.ds(s * RPT, RPT)],
                        deg_hbm.at[c, pl.ds(s * RPT, RPT)])

    return k(table, nid_pad, dst2w, ew2w, zrow)


WEA = 128
NWIN_A = (E_PAD // NW) // WEA
CWIN = 40
NW0, NW1 = 120, 40


def _sc_agg(hs, src2d, dst2d, ew2d, zblk, d_eff):
    @functools.partial(
        pl.kernel,
        out_type=jax.ShapeDtypeStruct((NC, NP, D1), jnp.float32),
        mesh=_vmesh(),
        compiler_params=_no_layout_cp(),
        scratch_types=[
            pltpu.VMEM_SHARED((NP, D1), jnp.float32),
            pltpu.VMEM((CWIN, WEA), jnp.int32),
            pltpu.VMEM((CWIN, WEA), jnp.int32),
            pltpu.VMEM((CWIN, WEA), jnp.float32),
            pltpu.VMEM((WEA, D1), jnp.float32),
            pltpu.VMEM((WEA, D1), jnp.float32),
            pltpu.SemaphoreType.DMA,
            pltpu.SemaphoreType.DMA,
            pltpu.SemaphoreType.DMA,
            pltpu.SemaphoreType.DMA,
        ],
    )
    def k(hs_hbm, src_hbm, dst_hbm, ew_hbm, zblk_hbm, out_hbm,
          acc, sbuf, dbuf, ebuf, in0, in1, g0, g1, s0, s1):
        c = lax.axis_index("c")
        s = lax.axis_index("s")
        w = c * NS + s

        def scale(buf, erow):
            @pl.loop(0, WEA, step=4)
            def _(i):
                for u in range(4):
                    iu = i + u
                    es = plsc.load_gather(
                        erow, [jnp.full((L,), iu, jnp.int32)])
                    for kk in range(d_eff // L):
                        sl = pl.ds(kk * L, L)
                        buf[iu, sl] = buf[iu, sl] * es

        q4 = RPT // 4
        for q in range(4):
            pltpu.make_async_copy(zblk_hbm.at[c, pl.ds(q * q4, q4)],
                                  acc.at[pl.ds(s * RPT + q * q4, q4)],
                                  s0).start()
        for q in range(4):
            pltpu.make_async_copy(zblk_hbm.at[c, pl.ds(q * q4, q4)],
                                  acc.at[pl.ds(s * RPT + q * q4, q4)],
                                  s0).wait()
        plsc.subcore_barrier()

        nch = jnp.where(c == 0, NW0 // CWIN, NW1 // CWIN)
        woff = jnp.where(c == 0, s * NW0, NS * NW0 + s * NW1)

        @pl.loop(0, nch)
        def _(ch):
            cbase = woff + ch * CWIN
            pltpu.sync_copy(src_hbm.at[pl.ds(cbase, CWIN)], sbuf)
            pltpu.sync_copy(dst_hbm.at[pl.ds(cbase, CWIN)], dbuf)
            pltpu.sync_copy(ew_hbm.at[pl.ds(cbase, CWIN)], ebuf)

            pltpu.make_async_copy(hs_hbm.at[sbuf.at[0]], in0, g0).start()

            @pl.loop(0, CWIN, step=2)
            def _(j):
                @pl.when(j > 0)
                def _():
                    pltpu.make_async_copy(in1, acc.at[dbuf.at[j - 1]],
                                          s1).wait()
                pltpu.make_async_copy(hs_hbm.at[sbuf.at[j + 1]], in1,
                                      g1).start()

                pltpu.make_async_copy(hs_hbm.at[sbuf.at[j]], in0, g0).wait()
                scale(in0, ebuf.at[j])
                pltpu.make_async_copy(in0, acc.at[dbuf.at[j]],
                                      s0).start(add=True)

                pltpu.make_async_copy(hs_hbm.at[sbuf.at[j + 1]], in1,
                                      g1).wait()
                scale(in1, ebuf.at[j + 1])

                pltpu.make_async_copy(in0, acc.at[dbuf.at[j]], s0).wait()

                @pl.when(j + 2 < CWIN)
                def _():
                    pltpu.make_async_copy(hs_hbm.at[sbuf.at[j + 2]], in0,
                                          g0).start()

                pltpu.make_async_copy(in1, acc.at[dbuf.at[j + 1]],
                                      s1).start(add=True)

            pltpu.make_async_copy(in1, acc.at[dbuf.at[CWIN - 1]], s1).wait()

        plsc.subcore_barrier()
        for q in range(4):
            pltpu.make_async_copy(acc.at[pl.ds(s * RPT + q * q4, q4)],
                                  out_hbm.at[c, pl.ds(s * RPT + q * q4, q4)],
                                  s0).start()
        for q in range(4):
            pltpu.make_async_copy(acc.at[pl.ds(s * RPT + q * q4, q4)],
                                  out_hbm.at[c, pl.ds(s * RPT + q * q4, q4)],
                                  s0).wait()

    return k(hs, src2d, dst2d, ew2d, zblk)


def _sc_decode(z, si, di):
    pw = LBL // NW

    @functools.partial(
        pl.kernel,
        out_type=(jax.ShapeDtypeStruct((LBL, D1), jnp.float32),
                  jax.ShapeDtypeStruct((LBL, D1), jnp.float32)),
        mesh=_vmesh(),
        scratch_types=[
            pltpu.VMEM((LBL // NW, D1), jnp.float32),
            pltpu.VMEM((1, LBL // NW), jnp.int32),
        ],
    )
    def k(z_hbm, si_hbm, di_hbm, zs_hbm, zd_hbm, rows, ibuf):
        c = lax.axis_index("c")
        s = lax.axis_index("s")
        base = (c * NS + s) * pw

        pltpu.sync_copy(si_hbm.at[pl.ds(base, pw)], ibuf.at[0])
        pltpu.sync_copy(z_hbm.at[ibuf.at[0]], rows)
        pltpu.sync_copy(rows, zs_hbm.at[pl.ds(base, pw)])

        pltpu.sync_copy(di_hbm.at[pl.ds(base, pw)], ibuf.at[0])
        pltpu.sync_copy(z_hbm.at[ibuf.at[0]], rows)
        pltpu.sync_copy(rows, zd_hbm.at[pl.ds(base, pw)])

    return k(z, si, di)


_BR = 512


def _dis_from(deg_ref):
    return lax.rsqrt(deg_ref[:, 0:1] + deg_ref[:, 1:2] + 1.0)


def _tc_scale_matmul(x, deg_t, w1):
    def body(x_ref, deg_ref, w_ref, o_ref):
        dis = _dis_from(deg_ref)
        h = jnp.dot(x_ref[...], w_ref[...], preferred_element_type=jnp.float32)
        o_ref[...] = h * dis

    return pl.pallas_call(
        body,
        grid=(NP // _BR,),
        in_specs=[pl.BlockSpec((_BR, D1), lambda i: (i, 0)),
                  pl.BlockSpec((_BR, NC), lambda i: (i, 0)),
                  pl.BlockSpec((D1, D1), lambda i: (0, 0))],
        out_specs=pl.BlockSpec((_BR, D1), lambda i: (i, 0)),
        out_shape=jax.ShapeDtypeStruct((NP, D1), jnp.float32),
    )(x, deg_t, w1)


def _tc_mid(agg1, hs1, deg_t, b1, w2):
    def body(agg_ref, hs_ref, deg_ref, b_ref, w_ref, o_ref):
        dis = _dis_from(deg_ref)
        tot = agg_ref[0] + agg_ref[1] + hs_ref[...]
        h = jnp.maximum(tot * dis + b_ref[...], 0.0)
        hs2 = jnp.dot(h, w_ref[...], preferred_element_type=jnp.float32) * dis
        o_ref[...] = jnp.concatenate(
            [hs2, jnp.zeros((_BR, D1 - D2), jnp.float32)], axis=1)

    return pl.pallas_call(
        body,
        grid=(NP // _BR,),
        in_specs=[pl.BlockSpec((NC, _BR, D1), lambda i: (0, i, 0)),
                  pl.BlockSpec((_BR, D1), lambda i: (i, 0)),
                  pl.BlockSpec((_BR, NC), lambda i: (i, 0)),
                  pl.BlockSpec((1, D1), lambda i: (0, 0)),
                  pl.BlockSpec((D1, D2), lambda i: (0, 0))],
        out_specs=pl.BlockSpec((_BR, D1), lambda i: (i, 0)),
        out_shape=jax.ShapeDtypeStruct((NP, D1), jnp.float32),
    )(agg1, hs1, deg_t, b1, w2)


def _tc_final(agg2, hs2, deg_t, b2):
    def body(agg_ref, hs_ref, deg_ref, b_ref, o_ref):
        dis = _dis_from(deg_ref)
        tot = agg_ref[0] + agg_ref[1] + hs_ref[...]
        o_ref[...] = tot * dis + b_ref[...]

    return pl.pallas_call(
        body,
        grid=(NP // _BR,),
        in_specs=[pl.BlockSpec((NC, _BR, D1), lambda i: (0, i, 0)),
                  pl.BlockSpec((_BR, D1), lambda i: (i, 0)),
                  pl.BlockSpec((_BR, NC), lambda i: (i, 0)),
                  pl.BlockSpec((1, D1), lambda i: (0, 0))],
        out_specs=pl.BlockSpec((_BR, D1), lambda i: (i, 0)),
        out_shape=jax.ShapeDtypeStruct((NP, D1), jnp.float32),
    )(agg2, hs2, deg_t, b2)


def _tc_dot(zs, zd):
    def body(a_ref, b_ref, o_ref):
        o_ref[...] = jnp.sum(a_ref[:, :D2] * b_ref[:, :D2], axis=1,
                             keepdims=True)

    return pl.pallas_call(
        body,
        grid=(LBL // 1024,),
        in_specs=[pl.BlockSpec((1024, D1), lambda i: (i, 0)),
                  pl.BlockSpec((1024, D1), lambda i: (i, 0))],
        out_specs=pl.BlockSpec((1024, 1), lambda i: (i, 0)),
        out_shape=jax.ShapeDtypeStruct((LBL, 1), jnp.float32),
    )(zs, zd)


def kernel(n_id, edge_index, edge_attr, edge_label_index, table, W1, b1, W2, b2):
    i32 = jnp.int32
    nid_pad = jnp.concatenate(
        [n_id.astype(i32), jnp.zeros((NP - N,), i32)])
    src_pad = jnp.concatenate(
        [edge_index[0].astype(i32), jnp.zeros((E_PAD - E,), i32)])
    dst_pad = jnp.concatenate(
        [edge_index[1].astype(i32), jnp.zeros((E_PAD - E,), i32)])
    ew_pad = jnp.concatenate(
        [edge_attr.astype(jnp.float32), jnp.zeros((E_PAD - E,), jnp.float32)])
    si = edge_label_index[0].astype(i32)
    di = edge_label_index[1].astype(i32)

    zrow = jnp.zeros((RPT,), jnp.float32)
    zblk1 = jnp.zeros((NC, RPT, D1), jnp.float32)
    b2p = jnp.concatenate([b2.astype(jnp.float32),
                           jnp.zeros((D1 - D2,), jnp.float32)]).reshape(1, D1)

    src2d = src_pad.reshape(E_PAD // WEA, WEA)
    dst2d = dst_pad.reshape(E_PAD // WEA, WEA)
    ew2d = ew_pad.reshape(E_PAD // WEA, WEA)

    x, deg2 = _sc_pre(table, nid_pad, dst_pad.reshape(E_PAD // WE, WE),
                      ew_pad.reshape(E_PAD // WE, WE), zrow)
    deg_t = deg2.T

    hs1 = _tc_scale_matmul(x, deg_t, W1)
    agg1 = _sc_agg(hs1, src2d, dst2d, ew2d, zblk1, D1)
    hs2 = _tc_mid(agg1, hs1, deg_t, b1.reshape(1, D1), W2)
    agg2 = _sc_agg(hs2, src2d, dst2d, ew2d, zblk1, D2)
    z = _tc_final(agg2, hs2, deg_t, b2p)

    zs, zd = _sc_decode(z, si, di)
    return _tc_dot(zs, zd).reshape(LBL)
